# baseline (speedup 1.0000x reference)
import functools

import jax
import jax.numpy as jnp
from jax import lax
from jax.experimental import pallas as pl
from jax.experimental.pallas import tpu as pltpu
from jax.experimental.pallas import tpu_sc as plsc

F_FIELDS = 26
VOCAB = 100000
BATCH = 16384
D_CAT = 32
D_NUM = 64
D_QUAL = 24
D_TOT = D_CAT + D_NUM + D_QUAL

NC, NS = 2, 16
NW = NC * NS
B_W = BATCH // NW
CHUNK = 128
NCH = B_W // CHUNK
LANES = 16


def _sc_gather_body(idx_hbm, tab_hbm, out_hbm, idx_v, acc_v, buf_v, sem):
  wid = lax.axis_index("s") * NC + lax.axis_index("c")
  base = wid * B_W

  pltpu.sync_copy(idx_hbm.at[wid], idx_v)

  zeros = jnp.zeros((LANES,), jnp.float32)
  @pl.loop(0, B_W, unroll=8)
  def _zero(r):
    for h in range(D_CAT // LANES):
      acc_v[r, pl.ds(h * LANES, LANES)] = zeros

  def fire(f, slot):
    for c in range(NCH):
      pltpu.async_copy(
          tab_hbm.at[idx_v.at[f * NCH + c]],
          buf_v.at[slot].at[pl.ds(c * CHUNK, CHUNK)],
          sem.at[slot],
      )

  def drain(f, slot):
    for c in range(NCH):
      pltpu.make_async_copy(
          tab_hbm.at[idx_v.at[f * NCH + c]],
          buf_v.at[slot].at[pl.ds(c * CHUNK, CHUNK)],
          sem.at[slot],
      ).wait()

  fire(0, 0)

  @pl.loop(0, F_FIELDS, step=2)
  def _fields(f0):
    for b in range(2):
      f = f0 + b
      slot = b
      @pl.when(f + 1 < F_FIELDS)
      def _():
        fire(f + 1, 1 - slot)
      drain(f, slot)
      @pl.loop(0, B_W, unroll=8)
      def _acc(r):
        for h in range(D_CAT // LANES):
          v = buf_v[slot, r, pl.ds(h * LANES, LANES)]
          plsc.addupdate(acc_v.at[r, pl.ds(h * LANES, LANES)], v)

  pltpu.sync_copy(acc_v, out_hbm.at[pl.ds(base, B_W)])


def _sc_gather(flat_idx, flat_tab):
  mesh = plsc.VectorSubcoreMesh(
      core_axis_name="c", subcore_axis_name="s", num_cores=NC, num_subcores=NS)
  return pl.kernel(
      _sc_gather_body,
      out_type=jax.ShapeDtypeStruct((BATCH, D_CAT), jnp.float32),
      mesh=mesh,
      scratch_types=[
          pltpu.VMEM((F_FIELDS * NCH, CHUNK), jnp.int32),
          pltpu.VMEM((B_W, D_CAT), jnp.float32),
          pltpu.VMEM((2, B_W, D_CAT), jnp.float32),
          pltpu.SemaphoreType.DMA((2,)),
      ],
      compiler_params=pltpu.CompilerParams(use_tc_tiling_on_sc=False),
  )(flat_idx, flat_tab)


def _tc_body(ecats_ref, nums_ref, quals_ref, W1_ref, b1_ref, W2_ref,
             b2_ref, Wq_ref, bq_ref, gamma_ref, beta_ref, out_ref):
  ec = ecats_ref[...]
  h = jnp.maximum(
      jnp.dot(nums_ref[...], W1_ref[...],
              preferred_element_type=jnp.float32) + b1_ref[...], 0.0)
  e_num = jnp.dot(h, W2_ref[...],
                  preferred_element_type=jnp.float32) + b2_ref[...]
  e_qual = jnp.dot(quals_ref[...], Wq_ref[...],
                   preferred_element_type=jnp.float32) + bq_ref[...]

  s = (jnp.sum(ec, -1, keepdims=True) + jnp.sum(e_num, -1, keepdims=True)
       + jnp.sum(e_qual, -1, keepdims=True))
  sq = (jnp.sum(ec * ec, -1, keepdims=True)
        + jnp.sum(e_num * e_num, -1, keepdims=True)
        + jnp.sum(e_qual * e_qual, -1, keepdims=True))
  mu = s * (1.0 / D_TOT)
  var = sq * (1.0 / D_TOT) - mu * mu
  inv = lax.rsqrt(var + 1e-5)

  g = gamma_ref[...]
  bt = beta_ref[...]
  out_ref[:, 0:D_CAT] = ((ec - mu) * inv) * g[:, 0:D_CAT] + bt[:, 0:D_CAT]
  out_ref[:, D_CAT:D_CAT + D_NUM] = (
      ((e_num - mu) * inv) * g[:, D_CAT:D_CAT + D_NUM]
      + bt[:, D_CAT:D_CAT + D_NUM])
  out_ref[:, D_CAT + D_NUM:D_TOT] = (
      ((e_qual - mu) * inv) * g[:, D_CAT + D_NUM:D_TOT]
      + bt[:, D_CAT + D_NUM:D_TOT])


def _tc_dense(e_cats, nums, quals, W1, b1, W2, b2, Wq, bq, gamma, beta):
  BLK = 2048
  grid = (BATCH // BLK,)
  full = lambda shape: pl.BlockSpec(shape, lambda i: (0, 0))
  return pl.pallas_call(
      _tc_body,
      grid=grid,
      in_specs=[
          pl.BlockSpec((BLK, D_CAT), lambda i: (i, 0)),
          pl.BlockSpec((BLK, 64), lambda i: (i, 0)),
          pl.BlockSpec((BLK, D_QUAL), lambda i: (i, 0)),
          full((64, 64)),
          full((1, 64)),
          full((64, D_NUM)),
          full((1, D_NUM)),
          full((D_QUAL, D_QUAL)),
          full((1, D_QUAL)),
          full((1, D_TOT)),
          full((1, D_TOT)),
      ],
      out_specs=pl.BlockSpec((BLK, D_TOT), lambda i: (i, 0)),
      out_shape=jax.ShapeDtypeStruct((BATCH, D_TOT), jnp.float32),
  )(e_cats, nums, quals, W1, b1, W2, b2, Wq, bq, gamma, beta)


def kernel(cats, nums, quals, tables, W1, b1, W2, b2, Wq, bq, gamma, beta):
  cats = cats.astype(jnp.int32)
  offs = (jnp.arange(F_FIELDS, dtype=jnp.int32) * VOCAB)[:, None]
  flat_idx = (cats + offs).reshape(F_FIELDS, NW, NCH, CHUNK)
  flat_idx = flat_idx.transpose(1, 0, 2, 3).reshape(NW, F_FIELDS * NCH, CHUNK)
  flat_tab = tables.reshape(F_FIELDS * VOCAB, D_CAT)

  e_cats = _sc_gather(flat_idx, flat_tab)

  return _tc_dense(
      e_cats, nums, quals, W1, b1.reshape(1, -1), W2, b2.reshape(1, -1),
      Wq, bq.reshape(1, -1), gamma.reshape(1, -1), beta.reshape(1, -1))
